# Initial kernel scaffold; baseline (speedup 1.0000x reference)
import jax
import jax.numpy as jnp
from jax.experimental import pallas as pl


def kernel(z, embedding, ema_cluster_size, ema_embedding):
    raise NotImplementedError("write your pallas kernel here")


# speedup vs baseline: 1.1798x; 1.1798x over previous
import functools

import jax
import jax.numpy as jnp
from jax import lax
from jax.experimental import pallas as pl
from jax.experimental.pallas import tpu as pltpu
from jax.experimental.pallas import tpu_sc as plsc

NUM_CODES = 8192
DIM = 256
KT = 1024
NKT = NUM_CODES // KT
BETA = 0.25


def _dist_argmin_body(z_ref, e_ref, idx_ref, loss_ref, minv_ref, mini_ref,
                      n_batches, n_elem):
    b = pl.program_id(0)
    k = pl.program_id(1)
    zb = z_ref[0]
    e = e_ref[...]
    z2 = jnp.sum(zb * zb, axis=0, keepdims=True)
    e2 = jnp.sum(e * e, axis=1, keepdims=True)
    dot = lax.dot_general(e, zb, (((1,), (0,)), ((), ())),
                          preferred_element_type=jnp.float32)
    scores = (z2 - 2.0 * dot) + e2
    local_min = jnp.min(scores, axis=0, keepdims=True)
    ids = lax.broadcasted_iota(jnp.int32, scores.shape, 0) + k * KT
    local_idx = jnp.min(jnp.where(scores == local_min, ids, 2 ** 30),
                        axis=0, keepdims=True)

    @pl.when(k == 0)
    def _():
        minv_ref[...] = local_min
        mini_ref[...] = local_idx

    @pl.when(k > 0)
    def _():
        better = local_min < minv_ref[...]
        mini_ref[...] = jnp.where(better, local_idx, mini_ref[...])
        minv_ref[...] = jnp.where(better, local_min, minv_ref[...])

    @pl.when(k == NKT - 1)
    def _():
        idx_ref[0] = mini_ref[...]
        partial = jnp.sum(minv_ref[...], keepdims=True)
        prev = jnp.where(b == 0, 0.0, loss_ref[...])
        total = prev + partial
        loss_ref[...] = jnp.where(b == n_batches - 1,
                                  total * (BETA / n_elem), total)


def _dist_argmin(z3, embedding):
    n_b, _, tok = z3.shape
    n_elem = n_b * DIM * tok
    body = functools.partial(_dist_argmin_body, n_batches=n_b, n_elem=n_elem)
    return pl.pallas_call(
        body,
        grid=(n_b, NKT),
        in_specs=[
            pl.BlockSpec((1, DIM, tok), lambda b, k: (b, 0, 0)),
            pl.BlockSpec((KT, DIM), lambda b, k: (k, 0)),
        ],
        out_specs=[
            pl.BlockSpec((1, 1, tok), lambda b, k: (b, 0, 0)),
            pl.BlockSpec((1, 1), lambda b, k: (0, 0)),
        ],
        out_shape=[
            jax.ShapeDtypeStruct((n_b, 1, tok), jnp.int32),
            jax.ShapeDtypeStruct((1, 1), jnp.float32),
        ],
        scratch_shapes=[
            pltpu.VMEM((1, tok), jnp.float32),
            pltpu.VMEM((1, tok), jnp.int32),
        ],
    )(z3, embedding)


def _sc_gather(indices, table):
    n = indices.shape[0]
    info = plsc.get_sparse_core_info()
    nw = info.num_cores * info.num_subcores
    per_w = n // nw
    mesh = plsc.VectorSubcoreMesh(core_axis_name="c", subcore_axis_name="s")

    @functools.partial(
        pl.kernel,
        mesh=mesh,
        out_type=jax.ShapeDtypeStruct((n, DIM), jnp.float32),
        scratch_types=[
            pltpu.VMEM((per_w,), jnp.int32),
            pltpu.VMEM((per_w, DIM), jnp.float32),
            pltpu.SemaphoreType.DMA,
        ],
    )
    def gather_kernel(idx_hbm, table_hbm, out_hbm, idx_v, rows_v, sem):
        wid = lax.axis_index("s") * info.num_cores + lax.axis_index("c")
        base = wid * per_w
        pltpu.sync_copy(idx_hbm.at[pl.ds(base, per_w)], idx_v)
        pltpu.async_copy(table_hbm.at[idx_v], rows_v, sem).wait()
        pltpu.sync_copy(rows_v, out_hbm.at[pl.ds(base, per_w)])

    return gather_kernel(indices, table)


def kernel(z, embedding, ema_cluster_size, ema_embedding):
    del ema_cluster_size, ema_embedding
    b, d, h, w = z.shape
    tok = h * w
    z3 = z.reshape(b, d, tok)
    idx3, loss = _dist_argmin(z3, embedding)
    indices = idx3.reshape(b * tok)
    q_flat = _sc_gather(indices, embedding)
    quantized = jnp.transpose(q_flat.reshape(b, h, w, d), (0, 3, 1, 2))
    return (quantized, loss[0, 0], indices)


# speedup vs baseline: 1.2324x; 1.0446x over previous
import functools

import jax
import jax.numpy as jnp
from jax import lax
from jax.experimental import pallas as pl
from jax.experimental.pallas import tpu as pltpu
from jax.experimental.pallas import tpu_sc as plsc

NUM_CODES = 8192
DIM = 256
KT = 1024
NKT = NUM_CODES // KT
BETA = 0.25


def _dist_argmin_body(z_ref, e_ref, idx_ref, loss_ref, minv_ref, mini_ref,
                      n_batches, n_elem):
    b = pl.program_id(0)
    k = pl.program_id(1)
    zb = z_ref[0]
    e = e_ref[...]
    e2 = jnp.sum(e * e, axis=1, keepdims=True)
    dot = lax.dot_general(e, zb, (((1,), (0,)), ((), ())),
                          preferred_element_type=jnp.float32)
    scores = e2 - 2.0 * dot
    local_min = jnp.min(scores, axis=0, keepdims=True)
    ids = lax.broadcasted_iota(jnp.int32, scores.shape, 0) + k * KT
    local_idx = jnp.min(jnp.where(scores == local_min, ids, 2 ** 30),
                        axis=0, keepdims=True)

    @pl.when(k == 0)
    def _():
        minv_ref[...] = local_min
        mini_ref[...] = local_idx

    @pl.when(k > 0)
    def _():
        better = local_min < minv_ref[...]
        mini_ref[...] = jnp.where(better, local_idx, mini_ref[...])
        minv_ref[...] = jnp.where(better, local_min, minv_ref[...])

    @pl.when(k == NKT - 1)
    def _():
        idx_ref[0] = mini_ref[...]
        z2 = jnp.sum(zb * zb, axis=0, keepdims=True)
        partial = jnp.sum(minv_ref[...] + z2, keepdims=True)
        prev = jnp.where(b == 0, 0.0, loss_ref[...])
        total = prev + partial
        loss_ref[...] = jnp.where(b == n_batches - 1,
                                  total * (BETA / n_elem), total)


def _dist_argmin(z3, embedding):
    n_b, _, tok = z3.shape
    n_elem = n_b * DIM * tok
    body = functools.partial(_dist_argmin_body, n_batches=n_b, n_elem=n_elem)
    return pl.pallas_call(
        body,
        grid=(n_b, NKT),
        in_specs=[
            pl.BlockSpec((1, DIM, tok), lambda b, k: (b, 0, 0)),
            pl.BlockSpec((KT, DIM), lambda b, k: (k, 0)),
        ],
        out_specs=[
            pl.BlockSpec((1, 1, tok), lambda b, k: (b, 0, 0)),
            pl.BlockSpec((1, 1), lambda b, k: (0, 0)),
        ],
        out_shape=[
            jax.ShapeDtypeStruct((n_b, 1, tok), jnp.int32),
            jax.ShapeDtypeStruct((1, 1), jnp.float32),
        ],
        scratch_shapes=[
            pltpu.VMEM((1, tok), jnp.float32),
            pltpu.VMEM((1, tok), jnp.int32),
        ],
    )(z3, embedding)


def _sc_gather(indices, table):
    n = indices.shape[0]
    info = plsc.get_sparse_core_info()
    nw = info.num_cores * info.num_subcores
    per_w = n // nw
    mesh = plsc.VectorSubcoreMesh(core_axis_name="c", subcore_axis_name="s")

    @functools.partial(
        pl.kernel,
        mesh=mesh,
        out_type=jax.ShapeDtypeStruct((n, DIM), jnp.float32),
        scratch_types=[
            pltpu.VMEM((per_w,), jnp.int32),
            pltpu.VMEM((per_w, DIM), jnp.float32),
            pltpu.SemaphoreType.DMA,
        ],
    )
    def gather_kernel(idx_hbm, table_hbm, out_hbm, idx_v, rows_v, sem):
        wid = lax.axis_index("s") * info.num_cores + lax.axis_index("c")
        base = wid * per_w
        pltpu.sync_copy(idx_hbm.at[pl.ds(base, per_w)], idx_v)
        pltpu.async_copy(table_hbm.at[idx_v], rows_v, sem).wait()
        pltpu.sync_copy(rows_v, out_hbm.at[pl.ds(base, per_w)])

    return gather_kernel(indices, table)


def kernel(z, embedding, ema_cluster_size, ema_embedding):
    del ema_cluster_size, ema_embedding
    b, d, h, w = z.shape
    tok = h * w
    z3 = z.reshape(b, d, tok)
    idx3, loss = _dist_argmin(z3, embedding)
    indices = idx3.reshape(b * tok)
    q_flat = _sc_gather(indices, embedding)
    quantized = jnp.transpose(q_flat.reshape(b, h, w, d), (0, 3, 1, 2))
    return (quantized, loss[0, 0], indices)


# speedup vs baseline: 1.2617x; 1.0238x over previous
import functools

import jax
import jax.numpy as jnp
from jax import lax
from jax.experimental import pallas as pl
from jax.experimental.pallas import tpu as pltpu
from jax.experimental.pallas import tpu_sc as plsc

NUM_CODES = 8192
DIM = 256
KT = 1024
NKT = NUM_CODES // KT
CH = 8
NCH = KT // CH
BETA = 0.25


def _scores_into(s_ref, e, zb):
    e2 = jnp.sum(e * e, axis=1, keepdims=True)
    dot = lax.dot_general(e, zb, (((1,), (0,)), ((), ())),
                          preferred_element_type=jnp.float32)
    s_ref[...] = e2 - 2.0 * dot


def _scan_buf(s_ref, val_ref, cid_ref, k):
    base = (k - 1) * NCH

    def body(i, carry):
        rv, ri = carry
        s = s_ref[pl.ds(i * CH, CH), :]
        better = s < rv
        rv = jnp.minimum(rv, s)
        ri = jnp.where(better, base + i, ri)
        return rv, ri

    rv, ri = lax.fori_loop(0, NCH, body, (val_ref[...], cid_ref[...]),
                           unroll=4)
    val_ref[...] = rv
    cid_ref[...] = ri


def _lexmin(v1, r1, v2, r2):
    take = (v2 < v1) | ((v2 == v1) & (r2 < r1))
    return jnp.where(take, v2, v1), jnp.where(take, r2, r1)


def _dist_argmin_body(z_ref, e_ref, idx_ref, loss_ref,
                      sa_ref, sb_ref, val_ref, cid_ref,
                      n_batches, n_elem):
    b = pl.program_id(0)
    k = pl.program_id(1)
    zb = z_ref[0]
    par = lax.rem(k, 2)

    @pl.when((k < NKT) & (par == 0))
    def _():
        _scores_into(sa_ref, e_ref[...], zb)

    @pl.when((k < NKT) & (par == 1))
    def _():
        _scores_into(sb_ref, e_ref[...], zb)

    @pl.when(k == 1)
    def _():
        val_ref[...] = jnp.full(val_ref.shape, jnp.inf, jnp.float32)
        cid_ref[...] = jnp.zeros(cid_ref.shape, jnp.int32)

    @pl.when((k >= 1) & (par == 1))
    def _():
        _scan_buf(sa_ref, val_ref, cid_ref, k)

    @pl.when((k >= 2) & (par == 0))
    def _():
        _scan_buf(sb_ref, val_ref, cid_ref, k)

    @pl.when(k == NKT)
    def _():
        rv = val_ref[...]
        rows = cid_ref[...] * CH + lax.broadcasted_iota(
            jnp.int32, cid_ref.shape, 0)
        v, r = _lexmin(rv[0:4], rows[0:4], rv[4:8], rows[4:8])
        v, r = _lexmin(v[0:2], r[0:2], v[2:4], r[2:4])
        v, r = _lexmin(v[0:1], r[0:1], v[1:2], r[1:2])
        idx_ref[0] = r
        z2 = jnp.sum(zb * zb, axis=0, keepdims=True)
        partial = jnp.sum(v + z2, keepdims=True)
        prev = jnp.where(b == 0, 0.0, loss_ref[...])
        total = prev + partial
        loss_ref[...] = jnp.where(b == n_batches - 1,
                                  total * (BETA / n_elem), total)


def _dist_argmin(z3, embedding):
    n_b, _, tok = z3.shape
    n_elem = n_b * DIM * tok
    body = functools.partial(_dist_argmin_body, n_batches=n_b, n_elem=n_elem)
    return pl.pallas_call(
        body,
        grid=(n_b, NKT + 1),
        in_specs=[
            pl.BlockSpec((1, DIM, tok), lambda b, k: (b, 0, 0)),
            pl.BlockSpec((KT, DIM), lambda b, k: (jnp.minimum(k, NKT - 1), 0)),
        ],
        out_specs=[
            pl.BlockSpec((1, 1, tok), lambda b, k: (b, 0, 0)),
            pl.BlockSpec((1, 1), lambda b, k: (0, 0)),
        ],
        out_shape=[
            jax.ShapeDtypeStruct((n_b, 1, tok), jnp.int32),
            jax.ShapeDtypeStruct((1, 1), jnp.float32),
        ],
        scratch_shapes=[
            pltpu.VMEM((KT, tok), jnp.float32),
            pltpu.VMEM((KT, tok), jnp.float32),
            pltpu.VMEM((CH, tok), jnp.float32),
            pltpu.VMEM((CH, tok), jnp.int32),
        ],
    )(z3, embedding)


def _sc_gather(indices, table):
    n = indices.shape[0]
    info = plsc.get_sparse_core_info()
    nw = info.num_cores * info.num_subcores
    per_w = n // nw
    mesh = plsc.VectorSubcoreMesh(core_axis_name="c", subcore_axis_name="s")

    @functools.partial(
        pl.kernel,
        mesh=mesh,
        out_type=jax.ShapeDtypeStruct((n, DIM), jnp.float32),
        scratch_types=[
            pltpu.VMEM((per_w,), jnp.int32),
            pltpu.VMEM((per_w, DIM), jnp.float32),
            pltpu.SemaphoreType.DMA,
        ],
    )
    def gather_kernel(idx_hbm, table_hbm, out_hbm, idx_v, rows_v, sem):
        wid = lax.axis_index("s") * info.num_cores + lax.axis_index("c")
        base = wid * per_w
        pltpu.sync_copy(idx_hbm.at[pl.ds(base, per_w)], idx_v)
        pltpu.async_copy(table_hbm.at[idx_v], rows_v, sem).wait()
        pltpu.sync_copy(rows_v, out_hbm.at[pl.ds(base, per_w)])

    return gather_kernel(indices, table)


def kernel(z, embedding, ema_cluster_size, ema_embedding):
    del ema_cluster_size, ema_embedding
    b, d, h, w = z.shape
    tok = h * w
    z3 = z.reshape(b, d, tok)
    idx3, loss = _dist_argmin(z3, embedding)
    indices = idx3.reshape(b * tok)
    q_flat = _sc_gather(indices, embedding)
    quantized = jnp.transpose(q_flat.reshape(b, h, w, d), (0, 3, 1, 2))
    return (quantized, loss[0, 0], indices)


# speedup vs baseline: 1.3159x; 1.0429x over previous
import functools

import jax
import jax.numpy as jnp
from jax import lax
from jax.experimental import pallas as pl
from jax.experimental.pallas import tpu as pltpu
from jax.experimental.pallas import tpu_sc as plsc

NUM_CODES = 8192
DIM = 256
DAUG = DIM + 8
KT = 1024
NKT = NUM_CODES // KT
CH = 8
NCH = KT // CH
BETA = 0.25


def _bf16r(x):
    return x.astype(jnp.bfloat16).astype(jnp.float32)


def _lexmin(v1, r1, v2, r2):
    take = (v2 < v1) | ((v2 == v1) & (r2 < r1))
    return jnp.where(take, v2, v1), jnp.where(take, r2, r1)


def _dist_argmin_body(z_ref, e_ref, idx_ref, loss_ref,
                      eaug_ref, zaug_ref, val_ref, cid_ref,
                      n_batches, n_elem):
    b = pl.program_id(0)
    k = pl.program_id(1)
    tok = z_ref.shape[2]

    @pl.when(k == 0)
    def _():
        zaug_ref[0:DIM, :] = z_ref[0]
        pad = lax.broadcasted_iota(jnp.int32, (DAUG - DIM, tok), 0)
        zaug_ref[DIM:DAUG, :] = jnp.where(pad < 3, 1.0, 0.0)

    @pl.when(b == 0)
    def _():
        e = e_ref[...]
        e2 = jnp.sum(e * e, axis=1, keepdims=True)
        p0 = _bf16r(e2)
        r1 = e2 - p0
        p1 = _bf16r(r1)
        p2 = r1 - p1
        zcols = jnp.zeros((KT, DAUG - DIM - 3), jnp.float32)
        eaug_ref[pl.ds(k * KT, KT), :] = jnp.concatenate(
            [-2.0 * e, p0, p1, p2, zcols], axis=1)

    ea = eaug_ref[pl.ds(k * KT, KT), :]
    scores = lax.dot_general(ea, zaug_ref[...], (((1,), (0,)), ((), ())),
                             preferred_element_type=jnp.float32)

    rv = scores[0:CH]
    ri = jnp.full((CH, tok), k * NCH, jnp.int32)
    for i in range(1, NCH):
        s = scores[i * CH:(i + 1) * CH]
        better = s < rv
        rv = jnp.minimum(rv, s)
        ri = jnp.where(better, k * NCH + i, ri)

    @pl.when(k == 0)
    def _():
        val_ref[...] = rv
        cid_ref[...] = ri

    @pl.when(k > 0)
    def _():
        better = rv < val_ref[...]
        cid_ref[...] = jnp.where(better, ri, cid_ref[...])
        val_ref[...] = jnp.minimum(val_ref[...], rv)

    @pl.when(k == NKT - 1)
    def _():
        fv = val_ref[...]
        rows = cid_ref[...] * CH + lax.broadcasted_iota(
            jnp.int32, (CH, tok), 0)
        v, r = _lexmin(fv[0:4], rows[0:4], fv[4:8], rows[4:8])
        v, r = _lexmin(v[0:2], r[0:2], v[2:4], r[2:4])
        v, r = _lexmin(v[0:1], r[0:1], v[1:2], r[1:2])
        idx_ref[0] = r
        zb = z_ref[0]
        z2 = jnp.sum(zb * zb, axis=0, keepdims=True)
        partial = jnp.sum(v + z2, keepdims=True)
        prev = jnp.where(b == 0, 0.0, loss_ref[...])
        total = prev + partial
        loss_ref[...] = jnp.where(b == n_batches - 1,
                                  total * (BETA / n_elem), total)


def _dist_argmin(z3, embedding):
    n_b, _, tok = z3.shape
    n_elem = n_b * DIM * tok
    body = functools.partial(_dist_argmin_body, n_batches=n_b, n_elem=n_elem)
    return pl.pallas_call(
        body,
        grid=(n_b, NKT),
        in_specs=[
            pl.BlockSpec((1, DIM, tok), lambda b, k: (b, 0, 0)),
            pl.BlockSpec((KT, DIM), lambda b, k: (k, 0)),
        ],
        out_specs=[
            pl.BlockSpec((1, 1, tok), lambda b, k: (b, 0, 0)),
            pl.BlockSpec((1, 1), lambda b, k: (0, 0)),
        ],
        out_shape=[
            jax.ShapeDtypeStruct((n_b, 1, tok), jnp.int32),
            jax.ShapeDtypeStruct((1, 1), jnp.float32),
        ],
        scratch_shapes=[
            pltpu.VMEM((NUM_CODES, DAUG), jnp.float32),
            pltpu.VMEM((DAUG, tok), jnp.float32),
            pltpu.VMEM((CH, tok), jnp.float32),
            pltpu.VMEM((CH, tok), jnp.int32),
        ],
    )(z3, embedding)


def _sc_gather(indices, table):
    n = indices.shape[0]
    info = plsc.get_sparse_core_info()
    nw = info.num_cores * info.num_subcores
    per_w = n // nw
    mesh = plsc.VectorSubcoreMesh(core_axis_name="c", subcore_axis_name="s")

    @functools.partial(
        pl.kernel,
        mesh=mesh,
        out_type=jax.ShapeDtypeStruct((n, DIM), jnp.float32),
        scratch_types=[
            pltpu.VMEM((per_w,), jnp.int32),
            pltpu.VMEM((per_w, DIM), jnp.float32),
            pltpu.SemaphoreType.DMA,
        ],
    )
    def gather_kernel(idx_hbm, table_hbm, out_hbm, idx_v, rows_v, sem):
        wid = lax.axis_index("s") * info.num_cores + lax.axis_index("c")
        base = wid * per_w
        pltpu.sync_copy(idx_hbm.at[pl.ds(base, per_w)], idx_v)
        pltpu.async_copy(table_hbm.at[idx_v], rows_v, sem).wait()
        pltpu.sync_copy(rows_v, out_hbm.at[pl.ds(base, per_w)])

    return gather_kernel(indices, table)


def kernel(z, embedding, ema_cluster_size, ema_embedding):
    del ema_cluster_size, ema_embedding
    b, d, h, w = z.shape
    tok = h * w
    z3 = z.reshape(b, d, tok)
    idx3, loss = _dist_argmin(z3, embedding)
    indices = idx3.reshape(b * tok)
    q_flat = _sc_gather(indices, embedding)
    quantized = jnp.transpose(q_flat.reshape(b, h, w, d), (0, 3, 1, 2))
    return (quantized, loss[0, 0], indices)


# speedup vs baseline: 1.6208x; 1.2317x over previous
import functools

import jax
import jax.numpy as jnp
from jax import lax
from jax.experimental import pallas as pl
from jax.experimental.pallas import tpu as pltpu
from jax.experimental.pallas import tpu_sc as plsc

NUM_CODES = 8192
DIM = 256
KT = 1024
NKT = NUM_CODES // KT
CH = 8
NCH = KT // CH
BETA = 0.25


def _lexmin(v1, r1, v2, r2):
    take = (v2 < v1) | ((v2 == v1) & (r2 < r1))
    return jnp.where(take, v2, v1), jnp.where(take, r2, r1)


def _dist_argmin_body(z_ref, e_ref, idx_ref, loss_ref,
                      em2_ref, e2_ref, val_ref, cid_ref,
                      n_batches, n_elem):
    b = pl.program_id(0)
    k = pl.program_id(1)
    tok = z_ref.shape[2]

    @pl.when(b == 0)
    def _():
        e = e_ref[...]
        em2_ref[pl.ds(k * KT, KT), :] = -2.0 * e
        e2_ref[pl.ds(k * KT, KT), :] = jnp.sum(e * e, axis=1, keepdims=True)

    ea = em2_ref[pl.ds(k * KT, KT), :]
    dot = lax.dot_general(ea, z_ref[0], (((1,), (0,)), ((), ())),
                          preferred_element_type=jnp.float32)
    scores = dot + e2_ref[pl.ds(k * KT, KT), :]

    rv = scores[0:CH]
    ri = jnp.full((CH, tok), k * NCH, jnp.int32)
    for i in range(1, NCH):
        s = scores[i * CH:(i + 1) * CH]
        better = s < rv
        rv = jnp.minimum(rv, s)
        ri = jnp.where(better, k * NCH + i, ri)

    @pl.when(k == 0)
    def _():
        val_ref[...] = rv
        cid_ref[...] = ri

    @pl.when(k > 0)
    def _():
        better = rv < val_ref[...]
        cid_ref[...] = jnp.where(better, ri, cid_ref[...])
        val_ref[...] = jnp.minimum(val_ref[...], rv)

    @pl.when(k == NKT - 1)
    def _():
        fv = val_ref[...]
        rows = cid_ref[...] * CH + lax.broadcasted_iota(
            jnp.int32, (CH, tok), 0)
        v, r = _lexmin(fv[0:4], rows[0:4], fv[4:8], rows[4:8])
        v, r = _lexmin(v[0:2], r[0:2], v[2:4], r[2:4])
        v, r = _lexmin(v[0:1], r[0:1], v[1:2], r[1:2])
        idx_ref[0] = r
        zb = z_ref[0]
        z2 = jnp.sum(zb * zb, axis=0, keepdims=True)
        partial = jnp.sum(v + z2, keepdims=True)
        prev = jnp.where(b == 0, 0.0, loss_ref[...])
        total = prev + partial
        loss_ref[...] = jnp.where(b == n_batches - 1,
                                  total * (BETA / n_elem), total)


def _dist_argmin(z3, embedding):
    n_b, _, tok = z3.shape
    n_elem = n_b * DIM * tok
    body = functools.partial(_dist_argmin_body, n_batches=n_b, n_elem=n_elem)
    return pl.pallas_call(
        body,
        grid=(n_b, NKT),
        in_specs=[
            pl.BlockSpec((1, DIM, tok), lambda b, k: (b, 0, 0)),
            pl.BlockSpec((KT, DIM), lambda b, k: (k, 0)),
        ],
        out_specs=[
            pl.BlockSpec((1, 1, tok), lambda b, k: (b, 0, 0)),
            pl.BlockSpec((1, 1), lambda b, k: (0, 0)),
        ],
        out_shape=[
            jax.ShapeDtypeStruct((n_b, 1, tok), jnp.int32),
            jax.ShapeDtypeStruct((1, 1), jnp.float32),
        ],
        scratch_shapes=[
            pltpu.VMEM((NUM_CODES, DIM), jnp.float32),
            pltpu.VMEM((NUM_CODES, 1), jnp.float32),
            pltpu.VMEM((CH, tok), jnp.float32),
            pltpu.VMEM((CH, tok), jnp.int32),
        ],
    )(z3, embedding)


def _sc_gather(indices, table):
    n = indices.shape[0]
    info = plsc.get_sparse_core_info()
    nw = info.num_cores * info.num_subcores
    per_w = n // nw
    mesh = plsc.VectorSubcoreMesh(core_axis_name="c", subcore_axis_name="s")

    @functools.partial(
        pl.kernel,
        mesh=mesh,
        out_type=jax.ShapeDtypeStruct((n, DIM), jnp.float32),
        scratch_types=[
            pltpu.VMEM((per_w,), jnp.int32),
            pltpu.VMEM((per_w, DIM), jnp.float32),
            pltpu.SemaphoreType.DMA,
        ],
    )
    def gather_kernel(idx_hbm, table_hbm, out_hbm, idx_v, rows_v, sem):
        wid = lax.axis_index("s") * info.num_cores + lax.axis_index("c")
        base = wid * per_w
        pltpu.sync_copy(idx_hbm.at[pl.ds(base, per_w)], idx_v)
        pltpu.async_copy(table_hbm.at[idx_v], rows_v, sem).wait()
        pltpu.sync_copy(rows_v, out_hbm.at[pl.ds(base, per_w)])

    return gather_kernel(indices, table)


def kernel(z, embedding, ema_cluster_size, ema_embedding):
    del ema_cluster_size, ema_embedding
    b, d, h, w = z.shape
    tok = h * w
    z3 = z.reshape(b, d, tok)
    idx3, loss = _dist_argmin(z3, embedding)
    indices = idx3.reshape(b * tok)
    q_flat = _sc_gather(indices, embedding)
    quantized = jnp.transpose(q_flat.reshape(b, h, w, d), (0, 3, 1, 2))
    return (quantized, loss[0, 0], indices)


# speedup vs baseline: 1.6548x; 1.0210x over previous
import functools

import jax
import jax.numpy as jnp
from jax import lax
from jax.experimental import pallas as pl
from jax.experimental.pallas import tpu as pltpu
from jax.experimental.pallas import tpu_sc as plsc

NUM_CODES = 8192
DIM = 256
KT = 1024
NKT = NUM_CODES // KT
NJ = NKT // 2
CH = 8
NCH = KT // CH
BETA = 0.25


def _lexmin(v1, r1, v2, r2):
    take = (v2 < v1) | ((v2 == v1) & (r2 < r1))
    return jnp.where(take, v2, v1), jnp.where(take, r2, r1)


def _scan(s_ref, tile, tok):
    rv = s_ref[0:CH, :]
    ri = jnp.full((CH, tok), tile * NCH, jnp.int32)
    for i in range(1, NCH):
        s = s_ref[i * CH:(i + 1) * CH, :]
        better = s < rv
        rv = jnp.minimum(rv, s)
        ri = jnp.where(better, tile * NCH + i, ri)
    return rv, ri


def _merge(val_ref, cid_ref, rv, ri):
    better = rv < val_ref[...]
    cid_ref[...] = jnp.where(better, ri, cid_ref[...])
    val_ref[...] = jnp.minimum(val_ref[...], rv)


def _dist_argmin_body(z_ref, e_ref, idx_ref, loss_ref,
                      em2_ref, e2_ref, sa_ref, sb_ref, val_ref, cid_ref,
                      n_batches, n_elem):
    b = pl.program_id(0)
    j = pl.program_id(1)
    tok = z_ref.shape[2]
    jj = jnp.minimum(j, NJ - 1)
    t0 = 2 * jj
    t1 = 2 * jj + 1

    @pl.when((b == 0) & (j < NJ))
    def _():
        e = e_ref[...]
        em2_ref[pl.ds(t0 * KT, 2 * KT), :] = -2.0 * e
        e2_ref[pl.ds(t0 * KT, 2 * KT), :] = jnp.sum(e * e, axis=1,
                                                    keepdims=True)

    zb = z_ref[0]
    ea0 = em2_ref[pl.ds(t0 * KT, KT), :]
    dot0 = lax.dot_general(ea0, zb, (((1,), (0,)), ((), ())),
                           preferred_element_type=jnp.float32)
    sa_ref[...] = dot0 + e2_ref[pl.ds(t0 * KT, KT), :]
    ea1 = em2_ref[pl.ds(t1 * KT, KT), :]
    dot1 = lax.dot_general(ea1, zb, (((1,), (0,)), ((), ())),
                           preferred_element_type=jnp.float32)
    rvb, rib = _scan(sb_ref, 2 * j - 1, tok)
    sb_ref[...] = dot1 + e2_ref[pl.ds(t1 * KT, KT), :]
    rva, ria = _scan(sa_ref, 2 * j, tok)

    @pl.when(j == 0)
    def _():
        val_ref[...] = rva
        cid_ref[...] = ria

    @pl.when(j > 0)
    def _():
        _merge(val_ref, cid_ref, rvb, rib)
        _merge(val_ref, cid_ref, rva, ria)

    @pl.when(j == NJ)
    def _():
        fv = val_ref[...]
        rows = cid_ref[...] * CH + lax.broadcasted_iota(
            jnp.int32, (CH, tok), 0)
        v, r = _lexmin(fv[0:4], rows[0:4], fv[4:8], rows[4:8])
        v, r = _lexmin(v[0:2], r[0:2], v[2:4], r[2:4])
        v, r = _lexmin(v[0:1], r[0:1], v[1:2], r[1:2])
        idx_ref[0] = r
        z2 = jnp.sum(zb * zb, axis=0, keepdims=True)
        partial = jnp.sum(v + z2, keepdims=True)
        prev = jnp.where(b == 0, 0.0, loss_ref[...])
        total = prev + partial
        loss_ref[...] = jnp.where(b == n_batches - 1,
                                  total * (BETA / n_elem), total)


def _dist_argmin(z3, embedding):
    n_b, _, tok = z3.shape
    n_elem = n_b * DIM * tok
    body = functools.partial(_dist_argmin_body, n_batches=n_b, n_elem=n_elem)
    return pl.pallas_call(
        body,
        grid=(n_b, NJ + 1),
        in_specs=[
            pl.BlockSpec((1, DIM, tok), lambda b, j: (b, 0, 0)),
            pl.BlockSpec((2 * KT, DIM),
                         lambda b, j: (jnp.minimum(j, NJ - 1), 0)),
        ],
        out_specs=[
            pl.BlockSpec((1, 1, tok), lambda b, j: (b, 0, 0)),
            pl.BlockSpec((1, 1), lambda b, j: (0, 0)),
        ],
        out_shape=[
            jax.ShapeDtypeStruct((n_b, 1, tok), jnp.int32),
            jax.ShapeDtypeStruct((1, 1), jnp.float32),
        ],
        scratch_shapes=[
            pltpu.VMEM((NUM_CODES, DIM), jnp.float32),
            pltpu.VMEM((NUM_CODES, 1), jnp.float32),
            pltpu.VMEM((KT, tok), jnp.float32),
            pltpu.VMEM((KT, tok), jnp.float32),
            pltpu.VMEM((CH, tok), jnp.float32),
            pltpu.VMEM((CH, tok), jnp.int32),
        ],
    )(z3, embedding)


def _sc_gather(indices, table):
    n = indices.shape[0]
    info = plsc.get_sparse_core_info()
    nw = info.num_cores * info.num_subcores
    per_w = n // nw
    mesh = plsc.VectorSubcoreMesh(core_axis_name="c", subcore_axis_name="s")

    @functools.partial(
        pl.kernel,
        mesh=mesh,
        out_type=jax.ShapeDtypeStruct((n, DIM), jnp.float32),
        scratch_types=[
            pltpu.VMEM((per_w,), jnp.int32),
            pltpu.VMEM((per_w, DIM), jnp.float32),
            pltpu.SemaphoreType.DMA,
        ],
    )
    def gather_kernel(idx_hbm, table_hbm, out_hbm, idx_v, rows_v, sem):
        wid = lax.axis_index("s") * info.num_cores + lax.axis_index("c")
        base = wid * per_w
        pltpu.sync_copy(idx_hbm.at[pl.ds(base, per_w)], idx_v)
        pltpu.async_copy(table_hbm.at[idx_v], rows_v, sem).wait()
        pltpu.sync_copy(rows_v, out_hbm.at[pl.ds(base, per_w)])

    return gather_kernel(indices, table)


def kernel(z, embedding, ema_cluster_size, ema_embedding):
    del ema_cluster_size, ema_embedding
    b, d, h, w = z.shape
    tok = h * w
    z3 = z.reshape(b, d, tok)
    idx3, loss = _dist_argmin(z3, embedding)
    indices = idx3.reshape(b * tok)
    q_flat = _sc_gather(indices, embedding)
    quantized = jnp.transpose(q_flat.reshape(b, h, w, d), (0, 3, 1, 2))
    return (quantized, loss[0, 0], indices)


# speedup vs baseline: 1.6727x; 1.0108x over previous
import functools

import jax
import jax.numpy as jnp
from jax import lax
from jax.experimental import pallas as pl
from jax.experimental.pallas import tpu as pltpu
from jax.experimental.pallas import tpu_sc as plsc

NUM_CODES = 8192
DIM = 256
KT = 1024
NKT = NUM_CODES // KT
NJ = NKT // 2
CH = 8
NCH = KT // CH
BETA = 0.25


def _lexmin(v1, r1, v2, r2):
    take = (v2 < v1) | ((v2 == v1) & (r2 < r1))
    return jnp.where(take, v2, v1), jnp.where(take, r2, r1)


def _scan(s_ref, tile, tok):
    rv = s_ref[0:CH, :]
    ri = jnp.full((CH, tok), tile * NCH, jnp.int32)
    for i in range(1, NCH):
        s = s_ref[i * CH:(i + 1) * CH, :]
        better = s < rv
        rv = jnp.minimum(rv, s)
        ri = jnp.where(better, tile * NCH + i, ri)
    return rv, ri


def _merge(val_ref, cid_ref, rv, ri):
    better = rv < val_ref[...]
    cid_ref[...] = jnp.where(better, ri, cid_ref[...])
    val_ref[...] = jnp.minimum(val_ref[...], rv)


def _dist_argmin_body(z_ref, e_ref, idx_ref, loss_ref,
                      em2_ref, e2_ref, sa_ref, sb_ref, val_ref, cid_ref,
                      n_batches, n_elem):
    b = pl.program_id(0)
    j = pl.program_id(1)
    tok = z_ref.shape[2]
    jj = jnp.minimum(j, NJ - 1)
    t0 = 2 * jj
    t1 = 2 * jj + 1

    @pl.when((b == 0) & (j < NJ))
    def _():
        e = e_ref[...]
        em2_ref[pl.ds(t0 * KT, 2 * KT), :] = -2.0 * e
        e2_ref[pl.ds(t0 * KT, 2 * KT), :] = jnp.sum(e * e, axis=1,
                                                    keepdims=True)

    zb = z_ref[0]
    ea0 = em2_ref[pl.ds(t0 * KT, KT), :]
    dot0 = lax.dot_general(ea0, zb, (((1,), (0,)), ((), ())),
                           preferred_element_type=jnp.float32)
    sa_ref[...] = dot0 + e2_ref[pl.ds(t0 * KT, KT), :]
    ea1 = em2_ref[pl.ds(t1 * KT, KT), :]
    dot1 = lax.dot_general(ea1, zb, (((1,), (0,)), ((), ())),
                           preferred_element_type=jnp.float32)
    rvb, rib = _scan(sb_ref, 2 * j - 1, tok)
    sb_ref[...] = dot1 + e2_ref[pl.ds(t1 * KT, KT), :]
    rva, ria = _scan(sa_ref, 2 * j, tok)

    @pl.when(j == 0)
    def _():
        val_ref[...] = rva
        cid_ref[...] = ria

    @pl.when(j > 0)
    def _():
        _merge(val_ref, cid_ref, rvb, rib)
        _merge(val_ref, cid_ref, rva, ria)

    @pl.when(j == NJ)
    def _():
        fv = val_ref[...]
        rows = cid_ref[...] * CH + lax.broadcasted_iota(
            jnp.int32, (CH, tok), 0)
        v, r = _lexmin(fv[0:4], rows[0:4], fv[4:8], rows[4:8])
        v, r = _lexmin(v[0:2], r[0:2], v[2:4], r[2:4])
        v, r = _lexmin(v[0:1], r[0:1], v[1:2], r[1:2])
        idx_ref[0] = r
        z2 = jnp.sum(zb * zb, axis=0, keepdims=True)
        partial = jnp.sum(v + z2, keepdims=True)
        prev = jnp.where(b == 0, 0.0, loss_ref[...])
        total = prev + partial
        loss_ref[...] = jnp.where(b == n_batches - 1,
                                  total * (BETA / n_elem), total)


def _dist_argmin(z3, embedding):
    n_b, _, tok = z3.shape
    n_elem = n_b * DIM * tok
    body = functools.partial(_dist_argmin_body, n_batches=n_b, n_elem=n_elem)
    return pl.pallas_call(
        body,
        grid=(n_b, NJ + 1),
        in_specs=[
            pl.BlockSpec((1, DIM, tok), lambda b, j: (b, 0, 0)),
            pl.BlockSpec((2 * KT, DIM),
                         lambda b, j: (jnp.where(b == 0,
                                                 jnp.minimum(j, NJ - 1),
                                                 NJ - 1), 0)),
        ],
        out_specs=[
            pl.BlockSpec((1, 1, tok), lambda b, j: (b, 0, 0)),
            pl.BlockSpec((1, 1), lambda b, j: (0, 0)),
        ],
        out_shape=[
            jax.ShapeDtypeStruct((n_b, 1, tok), jnp.int32),
            jax.ShapeDtypeStruct((1, 1), jnp.float32),
        ],
        scratch_shapes=[
            pltpu.VMEM((NUM_CODES, DIM), jnp.float32),
            pltpu.VMEM((NUM_CODES, 1), jnp.float32),
            pltpu.VMEM((KT, tok), jnp.float32),
            pltpu.VMEM((KT, tok), jnp.float32),
            pltpu.VMEM((CH, tok), jnp.float32),
            pltpu.VMEM((CH, tok), jnp.int32),
        ],
    )(z3, embedding)


def _sc_gather(indices, table):
    n = indices.shape[0]
    info = plsc.get_sparse_core_info()
    nw = info.num_cores * info.num_subcores
    per_w = n // nw
    mesh = plsc.VectorSubcoreMesh(core_axis_name="c", subcore_axis_name="s")

    @functools.partial(
        pl.kernel,
        mesh=mesh,
        out_type=jax.ShapeDtypeStruct((n, DIM), jnp.float32),
        scratch_types=[
            pltpu.VMEM((per_w,), jnp.int32),
            pltpu.VMEM((per_w, DIM), jnp.float32),
            pltpu.SemaphoreType.DMA,
        ],
    )
    def gather_kernel(idx_hbm, table_hbm, out_hbm, idx_v, rows_v, sem):
        wid = lax.axis_index("s") * info.num_cores + lax.axis_index("c")
        base = wid * per_w
        pltpu.sync_copy(idx_hbm.at[pl.ds(base, per_w)], idx_v)
        pltpu.async_copy(table_hbm.at[idx_v], rows_v, sem).wait()
        pltpu.sync_copy(rows_v, out_hbm.at[pl.ds(base, per_w)])

    return gather_kernel(indices, table)


def kernel(z, embedding, ema_cluster_size, ema_embedding):
    del ema_cluster_size, ema_embedding
    b, d, h, w = z.shape
    tok = h * w
    z3 = z.reshape(b, d, tok)
    idx3, loss = _dist_argmin(z3, embedding)
    indices = idx3.reshape(b * tok)
    q_flat = _sc_gather(indices, embedding)
    quantized = jnp.transpose(q_flat.reshape(b, h, w, d), (0, 3, 1, 2))
    return (quantized, loss[0, 0], indices)
